# Initial kernel scaffold; baseline (speedup 1.0000x reference)
import functools

import jax
import jax.numpy as jnp
from jax import lax
from jax.experimental import pallas as pl
from jax.experimental.pallas import tpu as pltpu
import jax.experimental.pallas.tpu_sc as plsc

_B, _L = 4096, 200
_N = _B * _L
_D = 64
_M = 128
_NW = 32
_RPW = _N // _NW
_C = 1024
_KI = _C // 128
_NCH = _RPW // _C


def _sc_gather(x2, table):
    mesh = plsc.VectorSubcoreMesh(core_axis_name="c", subcore_axis_name="s")

    @functools.partial(
        pl.kernel,
        out_type=jax.ShapeDtypeStruct((_N, _D), jnp.float32),
        mesh=mesh,
        scratch_types=[
            pltpu.VMEM((_KI, 128), jnp.int32),
            pltpu.VMEM((_C, _D), jnp.float32),
            pltpu.SemaphoreType.DMA,
        ],
    )
    def k(x_hbm, tab_hbm, emb_hbm, idx_v, rows_v, sem):
        wid = lax.axis_index("s") * 2 + lax.axis_index("c")
        base = wid * _RPW

        def body(i, carry):
            b = base + i * _C
            pltpu.sync_copy(x_hbm.at[pl.ds(b // 128, _KI)], idx_v)
            descs = [
                pltpu.async_copy(
                    tab_hbm.at[idx_v.at[j]],
                    rows_v.at[pl.ds(j * 128, 128)],
                    sem,
                )
                for j in range(_KI)
            ]
            for d in descs:
                d.wait()
            pltpu.sync_copy(rows_v, emb_hbm.at[pl.ds(b, _C)])
            return carry

        lax.fori_loop(0, _NCH, body, 0)

    return k(x2, table)


def _ln_body(emb_ref, pos_ref, g_ref, b_ref, out_ref):
    emb = emb_ref[...]
    pos = pos_ref[...]
    g = g_ref[...][0]
    b = b_ref[...][0]
    s_p = jnp.sum(pos, axis=-1)
    q_p = jnp.sum(pos * pos, axis=-1)
    s_e = jnp.sum(emb, axis=-1)
    q_e = jnp.sum(emb * emb, axis=-1)
    mu = (s_e + s_p[None, :]) * (1.0 / _M)
    var = (q_e + q_p[None, :]) * (1.0 / _M) - mu * mu
    r = lax.rsqrt(var + 1e-5)
    mu = mu[..., None]
    r = r[..., None]
    eo = (emb - mu) * r * g[:_D] + b[:_D]
    po = (pos[None] - mu) * r * g[_D:] + b[_D:]
    out_ref[...] = jnp.concatenate([eo, po], axis=-1)


def _tc_ln(emb3, pos, g2d, b2d):
    BB = 8
    return pl.pallas_call(
        _ln_body,
        grid=(_B // BB,),
        in_specs=[
            pl.BlockSpec((BB, _L, _D), lambda i: (i, 0, 0)),
            pl.BlockSpec((_L, _D), lambda i: (0, 0)),
            pl.BlockSpec((1, _M), lambda i: (0, 0)),
            pl.BlockSpec((1, _M), lambda i: (0, 0)),
        ],
        out_specs=pl.BlockSpec((BB, _L, _M), lambda i: (i, 0, 0)),
        out_shape=jax.ShapeDtypeStruct((_B, _L, _M), jnp.float32),
    )(emb3, pos, g2d, b2d)


def kernel(x, table, pos_weight, ln_gamma, ln_beta):
    x2 = x.astype(jnp.int32).reshape(_N // 128, 128)
    emb = _sc_gather(x2, table)
    pos = pos_weight[:, :_L].T
    out = _tc_ln(
        emb.reshape(_B, _L, _D),
        pos,
        ln_gamma.reshape(1, _M),
        ln_beta.reshape(1, _M),
    )
    return out


# speedup vs baseline: 3.2444x; 3.2444x over previous
import functools

import jax
import jax.numpy as jnp
from jax import lax
from jax.experimental import pallas as pl
from jax.experimental.pallas import tpu as pltpu
import jax.experimental.pallas.tpu_sc as plsc

_B, _L = 4096, 200
_N = _B * _L
_D = 64
_M = 128
_NW = 32
_RPW = _N // _NW
_C = 1024
_KI = _C // 128
_NCH = _RPW // _C


def _sc_gather(x2, table):
    mesh = plsc.VectorSubcoreMesh(core_axis_name="c", subcore_axis_name="s")

    @functools.partial(
        pl.kernel,
        out_type=jax.ShapeDtypeStruct((_N, _D), jnp.float32),
        mesh=mesh,
        scratch_types=[
            pltpu.VMEM((_KI, 128), jnp.int32),
            pltpu.VMEM((_C, _D), jnp.float32),
            pltpu.SemaphoreType.DMA,
        ],
        compiler_params=pltpu.CompilerParams(use_tc_tiling_on_sc=False),
    )
    def k(x_hbm, tab_hbm, emb_hbm, idx_v, rows_v, sem):
        wid = lax.axis_index("s") * 2 + lax.axis_index("c")
        base = wid * _RPW

        def body(i, carry):
            b = pl.multiple_of(base + i * _C, _C)
            pltpu.sync_copy(x_hbm.at[pl.ds(pl.multiple_of(b // 128, _KI), _KI)], idx_v)
            descs = [
                pltpu.async_copy(
                    tab_hbm.at[idx_v.at[j]],
                    rows_v.at[pl.ds(j * 128, 128)],
                    sem,
                )
                for j in range(_KI)
            ]
            for d in descs:
                d.wait()
            pltpu.sync_copy(rows_v, emb_hbm.at[pl.ds(b, _C)])
            return carry

        lax.fori_loop(0, _NCH, body, 0)

    return k(x2, table)


def _ln_body(emb_ref, pos_ref, g_ref, b_ref, out_ref):
    emb = emb_ref[...]
    pos = pos_ref[...]
    g = g_ref[...][0]
    b = b_ref[...][0]
    s_p = jnp.sum(pos, axis=-1)
    q_p = jnp.sum(pos * pos, axis=-1)
    s_e = jnp.sum(emb, axis=-1)
    q_e = jnp.sum(emb * emb, axis=-1)
    mu = (s_e + s_p[None, :]) * (1.0 / _M)
    var = (q_e + q_p[None, :]) * (1.0 / _M) - mu * mu
    r = lax.rsqrt(var + 1e-5)
    mu = mu[..., None]
    r = r[..., None]
    eo = (emb - mu) * r * g[:_D] + b[:_D]
    po = (pos[None] - mu) * r * g[_D:] + b[_D:]
    out_ref[...] = jnp.concatenate([eo, po], axis=-1)


def _tc_ln(emb3, pos, g2d, b2d):
    BB = 8
    return pl.pallas_call(
        _ln_body,
        grid=(_B // BB,),
        in_specs=[
            pl.BlockSpec((BB, _L, _D), lambda i: (i, 0, 0)),
            pl.BlockSpec((_L, _D), lambda i: (0, 0)),
            pl.BlockSpec((1, _M), lambda i: (0, 0)),
            pl.BlockSpec((1, _M), lambda i: (0, 0)),
        ],
        out_specs=pl.BlockSpec((BB, _L, _M), lambda i: (i, 0, 0)),
        out_shape=jax.ShapeDtypeStruct((_B, _L, _M), jnp.float32),
    )(emb3, pos, g2d, b2d)


def kernel(x, table, pos_weight, ln_gamma, ln_beta):
    x2 = x.astype(jnp.int32).reshape(_N // 128, 128)
    emb = _sc_gather(x2, table)
    pos = pos_weight[:, :_L].T
    out = _tc_ln(
        emb.reshape(_B, _L, _D),
        pos,
        ln_gamma.reshape(1, _M),
        ln_beta.reshape(1, _M),
    )
    return out


# speedup vs baseline: 7.4032x; 2.2818x over previous
import functools

import jax
import jax.numpy as jnp
from jax import lax
from jax.experimental import pallas as pl
from jax.experimental.pallas import tpu as pltpu
import jax.experimental.pallas.tpu_sc as plsc

_B, _L = 4096, 200
_N = _B * _L
_D = 64
_M = 128
_NW = 32
_RPW = _N // _NW
_C = 128
_NCH = _RPW // _C
_IR = _RPW // 128


def _fused(x2, table, posT, gamma, beta):
    mesh = plsc.VectorSubcoreMesh(core_axis_name="c", subcore_axis_name="s")

    @functools.partial(
        pl.kernel,
        out_type=jax.ShapeDtypeStruct((_N, _M), jnp.float32),
        mesh=mesh,
        scratch_types=[
            pltpu.VMEM((_IR, 128), jnp.int32),
            pltpu.VMEM((_L, _D), jnp.float32),
            pltpu.VMEM((_M,), jnp.float32),
            pltpu.VMEM((_M,), jnp.float32),
            pltpu.SMEM((_L,), jnp.float32),
            pltpu.SMEM((_L,), jnp.float32),
            pltpu.VMEM((_C, _D), jnp.float32),
            pltpu.VMEM((_C, _D), jnp.float32),
            pltpu.VMEM((_C, _M), jnp.float32),
            pltpu.VMEM((_C, _M), jnp.float32),
            pltpu.SemaphoreType.DMA,
            pltpu.SemaphoreType.DMA,
            pltpu.SemaphoreType.DMA,
            pltpu.SemaphoreType.DMA,
        ],
        compiler_params=pltpu.CompilerParams(
            use_tc_tiling_on_sc=False, needs_layout_passes=False),
    )
    def k(x_hbm, tab_hbm, pos_hbm, g_hbm, b_hbm, out_hbm,
          idx_all, P_v, g_v, b_v, Sp_v, Qp_v, rows0, rows1, outb0, outb1,
          gsem0, gsem1, wsem0, wsem1):
        wid = lax.axis_index("s") * 2 + lax.axis_index("c")
        base = pl.multiple_of(wid * _RPW, _C)

        pltpu.sync_copy(
            x_hbm.at[pl.ds(pl.multiple_of(wid * _IR, 8), _IR)], idx_all)
        pltpu.sync_copy(pos_hbm, P_v)
        pltpu.sync_copy(g_hbm, g_v)
        pltpu.sync_copy(b_hbm, b_v)

        def pstats(l, carry):
            p0 = P_v[l, pl.ds(0, 16)]
            p1 = P_v[l, pl.ds(16, 16)]
            p2 = P_v[l, pl.ds(32, 16)]
            p3 = P_v[l, pl.ds(48, 16)]
            Sp_v[l] = jnp.sum((p0 + p1) + (p2 + p3))
            Qp_v[l] = jnp.sum((p0 * p0 + p1 * p1) + (p2 * p2 + p3 * p3))
            return carry

        lax.fori_loop(0, _L, pstats, 0)

        gv = [g_v[pl.ds(t * 16, 16)] for t in range(8)]
        bv = [b_v[pl.ds(t * 16, 16)] for t in range(8)]

        rows = (rows0, rows1)
        outs = (outb0, outb1)
        gsems = (gsem0, gsem1)
        wsems = (wsem0, wsem1)

        def start_gather(i, c):
            pltpu.async_copy(tab_hbm.at[idx_all.at[i]], rows[c], gsems[c])

        def wait_gather(c):
            pltpu.make_async_copy(
                tab_hbm.at[idx_all.at[0]], rows[c], gsems[c]).wait()

        def wait_write(c):
            pltpu.make_async_copy(
                outs[c], out_hbm.at[pl.ds(0, _C)], wsems[c]).wait()

        def compute(c, i):
            r_ref = rows[c]
            o_ref = outs[c]
            l0 = lax.rem(i * _C, _L)

            @plsc.parallel_loop(0, _C, 1, unroll=4)
            def row(j):
                lj = l0 + j
                l = jnp.where(lj >= _L, lj - _L, lj)
                e = [r_ref[j, pl.ds(t * 16, 16)] for t in range(4)]
                p = [P_v[l, pl.ds(t * 16, 16)] for t in range(4)]
                s = jnp.sum((e[0] + e[1]) + (e[2] + e[3])) + Sp_v[l]
                q = jnp.sum((e[0] * e[0] + e[1] * e[1])
                            + (e[2] * e[2] + e[3] * e[3])) + Qp_v[l]
                mu = s * (1.0 / _M)
                var = q * (1.0 / _M) - mu * mu + 1e-5
                xb = lax.bitcast_convert_type(var, jnp.int32)
                seed = jnp.int32(0x5F3759DF) - lax.shift_right_logical(xb, 1)
                y = lax.bitcast_convert_type(seed, jnp.float32)
                hx = 0.5 * var
                y = y * (1.5 - hx * y * y)
                y = y * (1.5 - hx * y * y)
                y = y * (1.5 - hx * y * y)
                t1 = mu * y
                ybc = jnp.broadcast_to(y, (16,))
                tbc = jnp.broadcast_to(t1, (16,))
                for t in range(4):
                    o_ref[j, pl.ds(t * 16, 16)] = (
                        (e[t] * ybc - tbc) * gv[t] + bv[t])
                for t in range(4):
                    o_ref[j, pl.ds(_D + t * 16, 16)] = (
                        (p[t] * ybc - tbc) * gv[4 + t] + bv[4 + t])

        start_gather(0, 0)
        start_gather(1, 1)

        def step(ii, carry):
            for sub in range(2):
                i = ii * 2 + sub
                c = sub
                wait_gather(c)

                @pl.when(i >= 2)
                def _():
                    wait_write(c)

                compute(c, i)
                b = pl.multiple_of(base + i * _C, _C)
                pltpu.async_copy(outs[c], out_hbm.at[pl.ds(b, _C)], wsems[c])

                @pl.when(i + 2 < _NCH)
                def _():
                    start_gather(i + 2, c)

            return carry

        lax.fori_loop(0, _NCH // 2, step, 0)
        wait_write(0)
        wait_write(1)

    return k(x2, table, posT, gamma, beta)


def kernel(x, table, pos_weight, ln_gamma, ln_beta):
    x2 = x.astype(jnp.int32).reshape(_N // 128, 128)
    posT = pos_weight[:, :_L].T
    h = _fused(x2, table, posT, ln_gamma, ln_beta)
    return h.reshape(_B, _L, _M)


# speedup vs baseline: 8.2217x; 1.1106x over previous
import functools

import jax
import jax.numpy as jnp
from jax import lax
from jax.experimental import pallas as pl
from jax.experimental.pallas import tpu as pltpu
import jax.experimental.pallas.tpu_sc as plsc

_B, _L = 4096, 200
_N = _B * _L
_D = 64
_DA = 80
_M = 128
_V = 100000
_NW = 32
_RPW = _N // _NW
_C = 128
_NCH = _RPW // _C
_IR = _RPW // 128


def _prep_body(t_ref, out_ref):
    t = t_ref[...]
    s = jnp.sum(t, axis=-1, keepdims=True)
    q = jnp.sum(t * t, axis=-1, keepdims=True)
    pad = jnp.zeros((t.shape[0], _DA - _D - 2), jnp.float32)
    out_ref[...] = jnp.concatenate([t, s, q, pad], axis=-1)


def _tc_prep(table):
    R = 2000
    return pl.pallas_call(
        _prep_body,
        grid=(_V // R,),
        in_specs=[pl.BlockSpec((R, _D), lambda i: (i, 0))],
        out_specs=pl.BlockSpec((R, _DA), lambda i: (i, 0)),
        out_shape=jax.ShapeDtypeStruct((_V, _DA), jnp.float32),
    )(table)


def _pstats_body(p_ref, out_ref):
    p = p_ref[...]
    s = jnp.sum(p, axis=-1, keepdims=True)
    q = jnp.sum(p * p, axis=-1, keepdims=True)
    pad = jnp.zeros((p.shape[0], 14), jnp.float32)
    out_ref[...] = jnp.concatenate([s, q, pad], axis=-1)


def _tc_pstats(posT):
    return pl.pallas_call(
        _pstats_body,
        out_shape=jax.ShapeDtypeStruct((_L, 16), jnp.float32),
    )(posT)


def _fused(x2, aug_tab, posT, pst, gamma, beta):
    mesh = plsc.VectorSubcoreMesh(core_axis_name="c", subcore_axis_name="s")

    @functools.partial(
        pl.kernel,
        out_type=jax.ShapeDtypeStruct((_N, _M), jnp.float32),
        mesh=mesh,
        scratch_types=[
            pltpu.VMEM((_IR, 128), jnp.int32),
            pltpu.VMEM((_L, _D), jnp.float32),
            pltpu.VMEM((_M,), jnp.float32),
            pltpu.VMEM((_M,), jnp.float32),
            pltpu.VMEM((_L, 16), jnp.float32),
            pltpu.VMEM((_C, _DA), jnp.float32),
            pltpu.VMEM((_C, _DA), jnp.float32),
            pltpu.VMEM((_C, _M), jnp.float32),
            pltpu.VMEM((_C, _M), jnp.float32),
            pltpu.SemaphoreType.DMA,
            pltpu.SemaphoreType.DMA,
            pltpu.SemaphoreType.DMA,
            pltpu.SemaphoreType.DMA,
        ],
        compiler_params=pltpu.CompilerParams(
            use_tc_tiling_on_sc=False, needs_layout_passes=False),
    )
    def k(x_hbm, tab_hbm, pos_hbm, pst_hbm, g_hbm, b_hbm, out_hbm,
          idx_all, P_v, g_v, b_v, PST_v, rows0, rows1, outb0, outb1,
          gsem0, gsem1, wsem0, wsem1):
        wid = lax.axis_index("s") * 2 + lax.axis_index("c")
        base = pl.multiple_of(wid * _RPW, _C)

        pltpu.sync_copy(
            x_hbm.at[pl.ds(pl.multiple_of(wid * _IR, 8), _IR)], idx_all)
        pltpu.sync_copy(pos_hbm, P_v)
        pltpu.sync_copy(pst_hbm, PST_v)
        pltpu.sync_copy(g_hbm, g_v)
        pltpu.sync_copy(b_hbm, b_v)

        gv = [g_v[pl.ds(t * 16, 16)] for t in range(8)]
        bv = [b_v[pl.ds(t * 16, 16)] for t in range(8)]

        rows = (rows0, rows1)
        outs = (outb0, outb1)
        gsems = (gsem0, gsem1)
        wsems = (wsem0, wsem1)

        def start_gather(i, c):
            pltpu.async_copy(tab_hbm.at[idx_all.at[i]], rows[c], gsems[c])

        def wait_gather(c):
            pltpu.make_async_copy(
                tab_hbm.at[idx_all.at[0]], rows[c], gsems[c]).wait()

        def wait_write(c):
            pltpu.make_async_copy(
                outs[c], out_hbm.at[pl.ds(0, _C)], wsems[c]).wait()

        iota16 = lax.iota(jnp.int32, 16)
        c64 = jnp.full((16,), _D, jnp.int32)
        c65 = jnp.full((16,), _D + 1, jnp.int32)
        c0 = jnp.zeros((16,), jnp.int32)
        c1 = jnp.ones((16,), jnp.int32)
        cmagic = jnp.full((16,), 0x5F3759DF, jnp.int32)

        def compute(c, i):
            r_ref = rows[c]
            o_ref = outs[c]
            l0 = lax.rem(i * _C, _L)
            lvec0 = l0 + iota16
            lvec0 = jnp.where(lvec0 >= _L, lvec0 - _L, lvec0)

            def group(g, lvec):
                rvec = g * 16 + iota16
                S = plsc.load_gather(r_ref, [rvec, c64])
                Q = plsc.load_gather(r_ref, [rvec, c65])
                Sp = plsc.load_gather(PST_v, [lvec, c0])
                Qp = plsc.load_gather(PST_v, [lvec, c1])
                mu = (S + Sp) * (1.0 / _M)
                var = (Q + Qp) * (1.0 / _M) - mu * mu + 1e-5
                seed = cmagic - lax.shift_right_logical(
                    plsc.bitcast(var, jnp.int32), 1)
                y = plsc.bitcast(seed, jnp.float32)
                hx = 0.5 * var
                y = y * (1.5 - hx * y * y)
                y = y * (1.5 - hx * y * y)
                t1 = mu * y

                @plsc.parallel_loop(0, 16, 1, unroll=4,
                                    carry=jnp.zeros((16,), jnp.int32))
                def rowp(j, jv):
                    jj = g * 16 + j
                    lj = l0 + jj
                    l = jnp.where(lj >= _L, lj - _L, lj)
                    e = [r_ref[jj, pl.ds(t * 16, 16)] for t in range(4)]
                    p = [P_v[l, pl.ds(t * 16, 16)] for t in range(4)]
                    ybc = y.at[jv].get(mode="promise_in_bounds")
                    tbc = t1.at[jv].get(mode="promise_in_bounds")
                    for t in range(4):
                        o_ref[jj, pl.ds(t * 16, 16)] = (
                            (e[t] * ybc - tbc) * gv[t] + bv[t])
                    for t in range(4):
                        o_ref[jj, pl.ds(_D + t * 16, 16)] = (
                            (p[t] * ybc - tbc) * gv[4 + t] + bv[4 + t])
                    return jv + 1

                lv2 = lvec + 16
                return jnp.where(lv2 >= _L, lv2 - _L, lv2)

            lax.fori_loop(0, _C // 16, group, lvec0)

        start_gather(0, 0)
        start_gather(1, 1)

        def step(ii, carry):
            for sub in range(2):
                i = ii * 2 + sub
                c = sub
                wait_gather(c)

                @pl.when(i >= 2)
                def _():
                    wait_write(c)

                compute(c, i)
                b = pl.multiple_of(base + i * _C, _C)
                pltpu.async_copy(outs[c], out_hbm.at[pl.ds(b, _C)], wsems[c])

                @pl.when(i + 2 < _NCH)
                def _():
                    start_gather(i + 2, c)

            return carry

        lax.fori_loop(0, _NCH // 2, step, 0)
        wait_write(0)
        wait_write(1)

    return k(x2, aug_tab, posT, pst, gamma, beta)


def kernel(x, table, pos_weight, ln_gamma, ln_beta):
    x2 = x.astype(jnp.int32).reshape(_N // 128, 128)
    posT = pos_weight[:, :_L].T
    aug = _tc_prep(table)
    pst = _tc_pstats(posT)
    h = _fused(x2, aug, posT, pst, ln_gamma, ln_beta)
    return h.reshape(_B, _L, _M)
